# baseline (speedup 1.0000x reference)
import functools

import jax
import jax.numpy as jnp
from jax import lax
from jax.experimental import pallas as pl
from jax.experimental.pallas import tpu as pltpu
from jax.experimental.pallas import tpu_sc as plsc

EMBEDDING_DIM = 128
CHUNK = 128
NSLOTS = 5
LOOKAHEAD = 1
BUILD_BLK = 128


def _sc_gather(table, idx3):
    info = plsc.get_sparse_core_info()
    nc, ns = info.num_cores, info.num_subcores
    nw = nc * ns
    V = table.shape[0]
    B = idx3.shape[0] * idx3.shape[1] * CHUNK
    b_per_w = B // nw
    n_chunks = b_per_w // CHUNK
    v_per_tile = -(-V // (ns * BUILD_BLK)) * BUILD_BLK
    v_pad = v_per_tile * ns
    n_build = v_per_tile // BUILD_BLK
    assert b_per_w * nw == B and n_chunks * CHUNK == b_per_w

    mesh = plsc.VectorSubcoreMesh(core_axis_name="c", subcore_axis_name="s")

    @functools.partial(
        pl.kernel,
        out_type=[
            jax.ShapeDtypeStruct((B, EMBEDDING_DIM), jnp.float32),
            jax.ShapeDtypeStruct((2, v_pad, EMBEDDING_DIM), jnp.float32),
        ],
        mesh=mesh,
        scratch_types=[
            pltpu.VMEM((n_chunks, CHUNK), jnp.int32),
            pltpu.VMEM((NSLOTS, CHUNK, EMBEDDING_DIM), jnp.float32),
            pltpu.VMEM((BUILD_BLK, EMBEDDING_DIM), jnp.float32),
            [pltpu.SemaphoreType.DMA] * NSLOTS,
            [pltpu.SemaphoreType.DMA] * NSLOTS,
        ],
    )
    def gather_kernel(
        t0_hbm, idx_hbm, out_hbm, tbl_hbm, idx_v, rows_v, build_v, sg, so
    ):
        cid = lax.axis_index("c")
        sid = lax.axis_index("s")
        wid = sid * nc + cid
        base = wid * b_per_w
        my_tbl = tbl_hbm.at[cid]

        row0 = sid * v_per_tile
        perm = lax.iota(jnp.int32, 16) ^ 1
        for k in range(n_build):
            blk = pl.multiple_of(
                jnp.minimum(row0 + k * BUILD_BLK, V - BUILD_BLK), 8
            )
            pltpu.sync_copy(t0_hbm.at[pl.ds(blk, BUILD_BLK)], build_v)

            dnums = lax.GatherDimensionNumbers(
                offset_dims=(), collapsed_slice_dims=(0,), start_index_map=(0,)
            )

            def swap_row(i, carry):
                for j in range(EMBEDDING_DIM // 16):
                    sl = pl.ds(16 * j, 16)
                    v = build_v[i, sl]
                    build_v[i, sl] = lax.gather(
                        v,
                        perm[:, None],
                        dimension_numbers=dnums,
                        slice_sizes=(1,),
                        mode=lax.GatherScatterMode.PROMISE_IN_BOUNDS,
                    )
                return carry

            lax.fori_loop(0, BUILD_BLK, swap_row, 0)
            pltpu.sync_copy(build_v, my_tbl.at[pl.ds(blk, BUILD_BLK)])

        pltpu.sync_copy(idx_hbm.at[wid], idx_v)

        plsc.subcore_barrier()

        def issue_gather(q, r):
            pltpu.async_copy(my_tbl.at[idx_v.at[q]], rows_v.at[r], sg[r])

        def wait_gather(r):
            pltpu.make_async_copy(
                my_tbl.at[idx_v.at[0]], rows_v.at[r], sg[r]
            ).wait()

        def issue_out(g, r):
            pltpu.async_copy(
                rows_v.at[r], out_hbm.at[pl.ds(base + g * CHUNK, CHUNK)], so[r]
            )

        def wait_out(r):
            pltpu.make_async_copy(
                rows_v.at[r], out_hbm.at[pl.ds(base, CHUNK)], so[r]
            ).wait()

        for r in range(LOOKAHEAD):
            issue_gather(r, r)

        def step(g, r, first_group, last_group):
            wait_gather(r)
            issue_out(g, r)
            q = g + LOOKAHEAD
            if not last_group or r < NSLOTS - LOOKAHEAD:
                rq = (r + LOOKAHEAD) % NSLOTS
                if not (first_group and r < NSLOTS - LOOKAHEAD):
                    wait_out(rq)
                issue_gather(q, rq)

        for r in range(NSLOTS):
            step(r, r, True, False)

        n_groups = n_chunks // NSLOTS

        def body(j, carry):
            g0 = j * NSLOTS
            for r in range(NSLOTS):
                step(g0 + r, r, False, False)
            return carry

        lax.fori_loop(1, n_groups - 1, body, 0)

        g0 = (n_groups - 1) * NSLOTS
        for r in range(NSLOTS):
            step(g0 + r, r, False, True)

        for r in range(NSLOTS):
            wait_out(r)

    out, _ = gather_kernel(table, idx3)
    return out


def kernel(positions, sin_cos_cache):
    rows, cols = positions.shape
    B = rows * cols
    idx3 = positions.T.reshape(32, B // (32 * CHUNK), CHUNK).astype(jnp.int32)
    table = sin_cos_cache.reshape(sin_cos_cache.shape[0], EMBEDDING_DIM)
    out = _sc_gather(table, idx3)
    return out.reshape(cols, rows, EMBEDDING_DIM).transpose(1, 0, 2)


# speedup vs baseline: 1.1736x; 1.1736x over previous
import functools

import jax
import jax.numpy as jnp
from jax import lax
from jax.experimental import pallas as pl
from jax.experimental.pallas import tpu as pltpu
from jax.experimental.pallas import tpu_sc as plsc

EMBEDDING_DIM = 128
CHUNK = 128
NSLOTS = 5
LOOKAHEAD = 3
BUILD_BLK = 128


def _sc_gather(table, idx3):
    info = plsc.get_sparse_core_info()
    nc, ns = info.num_cores, info.num_subcores
    nw = nc * ns
    V = table.shape[0]
    B = idx3.shape[0] * idx3.shape[1] * CHUNK
    b_per_w = B // nw
    n_chunks = b_per_w // CHUNK
    v_per_tile = -(-V // (ns * BUILD_BLK)) * BUILD_BLK
    v_pad = v_per_tile * ns
    n_build = v_per_tile // BUILD_BLK
    assert b_per_w * nw == B and n_chunks * CHUNK == b_per_w

    mesh = plsc.VectorSubcoreMesh(core_axis_name="c", subcore_axis_name="s")

    @functools.partial(
        pl.kernel,
        out_type=[
            jax.ShapeDtypeStruct((B, EMBEDDING_DIM), jnp.float32),
            jax.ShapeDtypeStruct((2, v_pad, EMBEDDING_DIM), jnp.float32),
        ],
        mesh=mesh,
        scratch_types=[
            pltpu.VMEM((n_chunks, CHUNK), jnp.int32),
            pltpu.VMEM((NSLOTS, CHUNK, EMBEDDING_DIM), jnp.float32),
            pltpu.VMEM((BUILD_BLK, EMBEDDING_DIM), jnp.float32),
            [pltpu.SemaphoreType.DMA] * NSLOTS,
            [pltpu.SemaphoreType.DMA] * NSLOTS,
        ],
    )
    def gather_kernel(
        t0_hbm, idx_hbm, out_hbm, tbl_hbm, idx_v, rows_v, build_v, sg, so
    ):
        cid = lax.axis_index("c")
        sid = lax.axis_index("s")
        wid = sid * nc + cid
        base = wid * b_per_w
        my_tbl = tbl_hbm.at[cid]

        row0 = sid * v_per_tile
        perm = lax.iota(jnp.int32, 16) ^ 1
        for k in range(n_build):
            blk = pl.multiple_of(
                jnp.minimum(row0 + k * BUILD_BLK, V - BUILD_BLK), 8
            )
            pltpu.sync_copy(t0_hbm.at[pl.ds(blk, BUILD_BLK)], build_v)

            dnums = lax.GatherDimensionNumbers(
                offset_dims=(), collapsed_slice_dims=(0,), start_index_map=(0,)
            )

            def swap_row(i, carry):
                for j in range(EMBEDDING_DIM // 16):
                    sl = pl.ds(16 * j, 16)
                    v = build_v[i, sl]
                    build_v[i, sl] = lax.gather(
                        v,
                        perm[:, None],
                        dimension_numbers=dnums,
                        slice_sizes=(1,),
                        mode=lax.GatherScatterMode.PROMISE_IN_BOUNDS,
                    )
                return carry

            lax.fori_loop(0, BUILD_BLK, swap_row, 0)
            pltpu.sync_copy(build_v, my_tbl.at[pl.ds(blk, BUILD_BLK)])

        pltpu.sync_copy(idx_hbm.at[wid], idx_v)

        plsc.subcore_barrier()

        def issue_gather(q, r):
            pltpu.async_copy(my_tbl.at[idx_v.at[q]], rows_v.at[r], sg[r])

        def wait_gather(r):
            pltpu.make_async_copy(
                my_tbl.at[idx_v.at[0]], rows_v.at[r], sg[r]
            ).wait()

        def issue_out(g, r):
            pltpu.async_copy(
                rows_v.at[r], out_hbm.at[pl.ds(base + g * CHUNK, CHUNK)], so[r]
            )

        def wait_out(r):
            pltpu.make_async_copy(
                rows_v.at[r], out_hbm.at[pl.ds(base, CHUNK)], so[r]
            ).wait()

        for r in range(LOOKAHEAD):
            issue_gather(r, r)

        def step(g, r, first_group, last_group):
            wait_gather(r)
            issue_out(g, r)
            q = g + LOOKAHEAD
            if not last_group or r < NSLOTS - LOOKAHEAD:
                rq = (r + LOOKAHEAD) % NSLOTS
                if not (first_group and r < NSLOTS - LOOKAHEAD):
                    wait_out(rq)
                issue_gather(q, rq)

        for r in range(NSLOTS):
            step(r, r, True, False)

        n_groups = n_chunks // NSLOTS

        def body(j, carry):
            g0 = j * NSLOTS
            for r in range(NSLOTS):
                step(g0 + r, r, False, False)
            return carry

        lax.fori_loop(1, n_groups - 1, body, 0)

        g0 = (n_groups - 1) * NSLOTS
        for r in range(NSLOTS):
            step(g0 + r, r, False, True)

        for r in range(NSLOTS):
            wait_out(r)

    out, _ = gather_kernel(table, idx3)
    return out


def kernel(positions, sin_cos_cache):
    rows, cols = positions.shape
    B = rows * cols
    idx3 = positions.T.reshape(32, B // (32 * CHUNK), CHUNK).astype(jnp.int32)
    table = sin_cos_cache.reshape(sin_cos_cache.shape[0], EMBEDDING_DIM)
    out = _sc_gather(table, idx3)
    return out.reshape(cols, rows, EMBEDDING_DIM).transpose(1, 0, 2)


# speedup vs baseline: 1.2178x; 1.0376x over previous
import functools

import jax
import jax.numpy as jnp
from jax import lax
from jax.experimental import pallas as pl
from jax.experimental.pallas import tpu as pltpu
from jax.experimental.pallas import tpu_sc as plsc

EMBEDDING_DIM = 128
CHUNK = 128
NSLOTS = 5
LOOKAHEAD = 3
BUILD_BLK = 128


def _sc_gather(table, idx3):
    info = plsc.get_sparse_core_info()
    nc, ns = info.num_cores, info.num_subcores
    nw = nc * ns
    V = table.shape[0]
    B = idx3.shape[0] * idx3.shape[1] * CHUNK
    b_per_w = B // nw
    n_chunks = b_per_w // CHUNK
    v_per_tile = -(-V // (ns * BUILD_BLK)) * BUILD_BLK
    v_pad = v_per_tile * ns
    n_build = v_per_tile // BUILD_BLK
    assert b_per_w * nw == B and n_chunks * CHUNK == b_per_w

    mesh = plsc.VectorSubcoreMesh(core_axis_name="c", subcore_axis_name="s")

    @functools.partial(
        pl.kernel,
        out_type=[
            jax.ShapeDtypeStruct((B, EMBEDDING_DIM), jnp.float32),
            jax.ShapeDtypeStruct((2, v_pad, EMBEDDING_DIM), jnp.float32),
        ],
        mesh=mesh,
        scratch_types=[
            pltpu.VMEM((n_chunks, CHUNK), jnp.int32),
            pltpu.VMEM((NSLOTS, CHUNK, EMBEDDING_DIM), jnp.float32),
            [pltpu.SemaphoreType.DMA] * NSLOTS,
            [pltpu.SemaphoreType.DMA] * NSLOTS,
        ],
    )
    def gather_kernel(
        t0_hbm, idx_hbm, out_hbm, tbl_hbm, idx_v, rows_v, sg, so
    ):
        cid = lax.axis_index("c")
        sid = lax.axis_index("s")
        wid = sid * nc + cid
        base = wid * b_per_w
        my_tbl = tbl_hbm.at[cid]

        assert n_build <= NSLOTS
        row0 = sid * v_per_tile
        perm = lax.iota(jnp.int32, 16) ^ 1
        dnums = lax.GatherDimensionNumbers(
            offset_dims=(), collapsed_slice_dims=(0,), start_index_map=(0,)
        )

        def build_blk(k):
            return pl.multiple_of(
                jnp.minimum(row0 + k * BUILD_BLK, V - BUILD_BLK), 8
            )

        for k in range(n_build):
            pltpu.async_copy(
                t0_hbm.at[pl.ds(build_blk(k), BUILD_BLK)], rows_v.at[k], sg[k]
            )
        pltpu.sync_copy(idx_hbm.at[wid], idx_v)
        for k in range(n_build):
            pltpu.make_async_copy(
                t0_hbm.at[pl.ds(0, BUILD_BLK)], rows_v.at[k], sg[k]
            ).wait()

            def swap_rows(i, carry):
                for u in range(2):
                    for j in range(EMBEDDING_DIM // 16):
                        sl = pl.ds(16 * j, 16)
                        v = rows_v[k, 2 * i + u, sl]
                        rows_v[k, 2 * i + u, sl] = lax.gather(
                            v,
                            perm[:, None],
                            dimension_numbers=dnums,
                            slice_sizes=(1,),
                            mode=lax.GatherScatterMode.PROMISE_IN_BOUNDS,
                        )
                return carry

            lax.fori_loop(0, BUILD_BLK // 2, swap_rows, 0)
            pltpu.sync_copy(rows_v.at[k], my_tbl.at[pl.ds(build_blk(k), BUILD_BLK)])

        plsc.subcore_barrier()

        def issue_gather(q, r):
            pltpu.async_copy(my_tbl.at[idx_v.at[q]], rows_v.at[r], sg[r])

        def wait_gather(r):
            pltpu.make_async_copy(
                my_tbl.at[idx_v.at[0]], rows_v.at[r], sg[r]
            ).wait()

        def issue_out(g, r):
            pltpu.async_copy(
                rows_v.at[r], out_hbm.at[pl.ds(base + g * CHUNK, CHUNK)], so[r]
            )

        def wait_out(r):
            pltpu.make_async_copy(
                rows_v.at[r], out_hbm.at[pl.ds(base, CHUNK)], so[r]
            ).wait()

        for r in range(LOOKAHEAD):
            issue_gather(r, r)

        def step(g, r, first_group, last_group):
            wait_gather(r)
            issue_out(g, r)
            q = g + LOOKAHEAD
            if not last_group or r < NSLOTS - LOOKAHEAD:
                rq = (r + LOOKAHEAD) % NSLOTS
                if not (first_group and r < NSLOTS - LOOKAHEAD):
                    wait_out(rq)
                issue_gather(q, rq)

        for r in range(NSLOTS):
            step(r, r, True, False)

        n_groups = n_chunks // NSLOTS

        def body(j, carry):
            g0 = j * NSLOTS
            for r in range(NSLOTS):
                step(g0 + r, r, False, False)
            return carry

        lax.fori_loop(1, n_groups - 1, body, 0)

        g0 = (n_groups - 1) * NSLOTS
        for r in range(NSLOTS):
            step(g0 + r, r, False, True)

        for r in range(NSLOTS):
            wait_out(r)

    out, _ = gather_kernel(table, idx3)
    return out


def kernel(positions, sin_cos_cache):
    rows, cols = positions.shape
    B = rows * cols
    idx3 = positions.T.reshape(32, B // (32 * CHUNK), CHUNK).astype(jnp.int32)
    table = sin_cos_cache.reshape(sin_cos_cache.shape[0], EMBEDDING_DIM)
    out = _sc_gather(table, idx3)
    return out.reshape(cols, rows, EMBEDDING_DIM).transpose(1, 0, 2)
